# Initial kernel scaffold; baseline (speedup 1.0000x reference)
import jax
import jax.numpy as jnp
from jax.experimental import pallas as pl


def kernel(idx, targets, table):
    raise NotImplementedError("write your pallas kernel here")


# speedup vs baseline: 1.3477x; 1.3477x over previous
import jax
import jax.numpy as jnp
from jax import lax
from jax.experimental import pallas as pl
from jax.experimental.pallas import tpu as pltpu
from jax.experimental.pallas import tpu_sc as plsc

V = 1000
B, T = 1024, 50
N = B * T
NC, NS = 2, 16
NW = NC * NS
PER_W = N // NW
CHUNK = 80
NCH = PER_W // CHUNK
CLOOP = CHUNK // 16


def _lse_body(table_ref, lse_ref):
    t = table_ref[...]
    m = jnp.max(t, axis=1)
    s = jnp.sum(jnp.exp(t - m[:, None]), axis=1)
    lse_ref[...] = m + jnp.log(s)


_lse_call = pl.pallas_call(
    _lse_body,
    out_shape=jax.ShapeDtypeStruct((V,), jnp.float32),
)


def _loss_body(part_ref, out_ref):
    out_ref[...] = jnp.sum(part_ref[...]).reshape(1, 1) * (1.0 / N)


_loss_call = pl.pallas_call(
    _loss_body,
    out_shape=jax.ShapeDtypeStruct((1, 1), jnp.float32),
)


def _sc_body(idx_hbm, tgt_hbm, table_hbm, tflat_hbm, lse_hbm,
             logits_hbm, part_hbm,
             idx_v, tgt_v, fidx_v, g_v, lg_v, rows_v, acc_v, sem, sem2):
    wid = lax.axis_index("s") * NC + lax.axis_index("c")
    base = wid * PER_W

    pltpu.sync_copy(idx_hbm.at[pl.ds(base, PER_W)], idx_v)
    pltpu.sync_copy(tgt_hbm.at[pl.ds(base, PER_W)], tgt_v)

    def fidx_body(k, carry):
        o = k * 16
        i16 = idx_v[pl.ds(o, 16)]
        t16 = tgt_v[pl.ds(o, 16)]
        fidx_v[pl.ds(o, 16)] = i16 * V + t16
        return carry

    lax.fori_loop(0, PER_W // 16, fidx_body, 0)

    def elem_body(c, carry):
        o = c * CHUNK
        pltpu.async_copy(
            tflat_hbm.at[fidx_v.at[pl.ds(o, CHUNK)]],
            g_v.at[pl.ds(o, CHUNK)], sem2,
        ).wait()
        pltpu.async_copy(
            lse_hbm.at[idx_v.at[pl.ds(o, CHUNK)]],
            lg_v.at[pl.ds(o, CHUNK)], sem2,
        ).wait()
        return carry

    lax.fori_loop(0, NCH, elem_body, 0)

    def chunk_body(c, carry):
        off = c * CHUNK
        pltpu.async_copy(
            table_hbm.at[idx_v.at[pl.ds(off, CHUNK)]], rows_v, sem
        ).wait()
        pltpu.sync_copy(rows_v, logits_hbm.at[pl.ds(base + off, CHUNK)])
        return carry

    lax.fori_loop(0, NCH, chunk_body, 0)

    acc_v[...] = jnp.zeros((16,), jnp.float32)

    def acc_body(k, carry):
        o = k * 16
        acc_v[...] = acc_v[...] + (lg_v[pl.ds(o, 16)] - g_v[pl.ds(o, 16)])
        return carry

    lax.fori_loop(0, PER_W // 16, acc_body, 0)
    pltpu.sync_copy(acc_v, part_hbm.at[wid])


_sc_gather = pl.kernel(
    _sc_body,
    mesh=plsc.VectorSubcoreMesh(core_axis_name="c", subcore_axis_name="s"),
    compiler_params=pltpu.CompilerParams(use_tc_tiling_on_sc=False),
    out_type=[
        jax.ShapeDtypeStruct((N, V), jnp.float32),
        jax.ShapeDtypeStruct((NW, 16), jnp.float32),
    ],
    scratch_types=[
        pltpu.VMEM((PER_W,), jnp.int32),
        pltpu.VMEM((PER_W,), jnp.int32),
        pltpu.VMEM((PER_W,), jnp.int32),
        pltpu.VMEM((PER_W,), jnp.float32),
        pltpu.VMEM((PER_W,), jnp.float32),
        pltpu.VMEM((CHUNK, V), jnp.float32),
        pltpu.VMEM((16,), jnp.float32),
        pltpu.SemaphoreType.DMA,
        pltpu.SemaphoreType.DMA,
    ],
)


@jax.jit
def _impl(idx, targets, table):
    lse = _lse_call(table)
    idx_flat = idx.reshape(N).astype(jnp.int32)
    tgt_flat = targets.reshape(N).astype(jnp.int32)
    tflat = jnp.pad(table.reshape(V * V), (0, 8))
    logits_flat, parts = _sc_gather(idx_flat, tgt_flat, table, tflat, lse)
    loss = _loss_call(parts)
    return logits_flat.reshape(B, T, V), loss.reshape(())


def kernel(idx, targets, table):
    return _impl(idx, targets, table)
